# Initial kernel scaffold; baseline (speedup 1.0000x reference)
import jax
import jax.numpy as jnp
from jax.experimental import pallas as pl


def kernel(x, edge_index, params):
    raise NotImplementedError("write your pallas kernel here")


# speedup vs baseline: 44.6790x; 44.6790x over previous
import functools

import jax
import jax.numpy as jnp
from jax import lax
from jax.experimental import pallas as pl
from jax.experimental.pallas import tpu as pltpu
from jax.experimental.pallas import tpu_sc as plsc

N = 10000
E = 320000
F = 64
H = 8
NW = 32
EW = E // NW
CH = 80
NCH = EW // CH
NT = 16
RT = N // NT
WD = 160

_f32 = jnp.float32


def _headsum_mat():
    j = lax.broadcasted_iota(jnp.int32, (F, H), 0)
    h = lax.broadcasted_iota(jnp.int32, (F, H), 1)
    return jnp.where(j // 8 == h, 1.0, 0.0).astype(_f32)


def _lrelu(z):
    return jnp.maximum(z, 0.2 * z)


def _elu(z):
    return jnp.where(z > 0, z, jnp.exp(jnp.minimum(z, 0.0)) - 1.0)


def _bn(r, g, b):
    mu = jnp.mean(r, axis=0, keepdims=True)
    d = r - mu
    var = jnp.mean(d * d, axis=0, keepdims=True)
    return d * lax.rsqrt(var + 1e-5) * g + b


def _dotT(a, w):
    return lax.dot_general(a, w, (((1,), (1,)), ((), ())),
                           preferred_element_type=_f32)


def _expand(e, S):
    return lax.dot_general(e, S, (((1,), (1,)), ((), ())),
                           preferred_element_type=_f32)


def _pre_body(x_ref, w1, g1, b1, w2, g2, b2, o_ref):
    z = _dotT(x_ref[...], w1[...])
    z = _elu(_bn(z, g1[...], b1[...]))
    z = _dotT(z, w2[...])
    o_ref[...] = _elu(_bn(z, g2[...], b2[...]))


def _pre(x, w1, g1, b1, w2, g2, b2):
    return pl.pallas_call(
        _pre_body,
        out_shape=jax.ShapeDtypeStruct((N, F), _f32),
    )(x, w1, g1, b1, w2, g2, b2)


def _tables_body(hl_ref, hr_ref, wlL, wrL, wlR, wrR, attL, attR, t1_ref, t2_ref):
    S = _headsum_mat()
    hl = hl_ref[...]
    hr = hr_ref[...]
    xlL = _dotT(hl, wlL[...])
    xrL = _dotT(hl, wrL[...])
    xlR = _dotT(hr, wlR[...])
    xrR = _dotT(hr, wrR[...])
    mL = jnp.dot(_lrelu(xlL + xrL) * attL[...], S, preferred_element_type=_f32)
    mR = jnp.dot(_lrelu(xlR + xrR) * attR[...], S, preferred_element_type=_f32)
    t1_ref[...] = jnp.concatenate([xlL, xlR], axis=1)
    t2_ref[...] = jnp.concatenate([xrL, mL, xrR, mR], axis=1)


def _tables(hl, hr, wlL, wrL, wlR, wrR, attL, attR):
    return pl.pallas_call(
        _tables_body,
        out_shape=[jax.ShapeDtypeStruct((N, 128), _f32),
                   jax.ShapeDtypeStruct((N, 144), _f32)],
    )(hl, hr, wlL, wrL, wlR, wrR, attL, attR)


BE = 2000


def _edge_body(g1_ref, g2_ref, v_ref, attL, attR, w_ref):
    S = _headsum_mat()
    g1 = g1_ref[...]
    g2 = g2_ref[...]
    v = v_ref[...]
    pad = jnp.zeros((BE, 7), _f32)

    def track(xl, xr, m, att):
        s = _lrelu(xl + xr)
        logit = jnp.dot(s * att, S, preferred_element_type=_f32)
        ex = jnp.exp(logit - m) * v
        w = _expand(ex, S) * xl
        return w, ex

    wL, exL = track(g1[:, 0:64], g2[:, 0:64], g2[:, 64:72], attL[...])
    wR, exR = track(g1[:, 64:128], g2[:, 72:136], g2[:, 136:144], attR[...])
    w_ref[...] = jnp.concatenate([wL, exL, v, pad, wR, exR, v, pad], axis=1)


def _edge(G1, G2, validf, attL, attR):
    return pl.pallas_call(
        _edge_body,
        grid=(E // BE,),
        in_specs=[
            pl.BlockSpec((BE, 128), lambda i: (i, 0)),
            pl.BlockSpec((BE, 144), lambda i: (i, 0)),
            pl.BlockSpec((BE, 1), lambda i: (i, 0)),
            pl.BlockSpec((1, F), lambda i: (0, 0)),
            pl.BlockSpec((1, F), lambda i: (0, 0)),
        ],
        out_specs=pl.BlockSpec((BE, WD), lambda i: (i, 0)),
        out_shape=jax.ShapeDtypeStruct((E, WD), _f32),
    )(G1, G2, validf, attL, attR)


def _combine_body(mean_aggr, acc_ref, xl_ref, h_ref, g_ref, b_ref, o_ref):
    S = _headsum_mat()
    s = acc_ref[0:N, :] + acc_ref[N:2 * N, :]
    num = s[:, 0:64] + xl_ref[...]
    den = s[:, 64:72] + 1.0
    g = num / _expand(den, S)
    if mean_aggr:
        deg = s[:, 72:73] + 1.0
        g = g / jnp.maximum(deg, 1.0)
    r = g + h_ref[...]
    o_ref[...] = _elu(_bn(r, g_ref[...], b_ref[...]))


def _combine(acc80, xl, h, g, b, mean_aggr):
    return pl.pallas_call(
        functools.partial(_combine_body, mean_aggr),
        out_shape=jax.ShapeDtypeStruct((N, F), _f32),
    )(acc80, xl, h, g, b)


def _post_body(hl_ref, hr_ref, w1, b1, w2, b2, w3, b3, w4, b4, o_ref):
    o = jnp.concatenate([hl_ref[...], hr_ref[...]], axis=1)
    o = _elu(_dotT(o, w1[...]) + b1[...])
    o = _elu(_dotT(o, w2[...]) + b2[...])
    o = _elu(_dotT(o, w3[...]) + b3[...])
    o_ref[...] = _dotT(o, w4[...]) + b4[...]


def _post(hl, hr, w1, b1, w2, b2, w3, b3, w4, b4):
    return pl.pallas_call(
        _post_body,
        out_shape=jax.ShapeDtypeStruct((N, 2), _f32),
    )(hl, hr, w1, b1, w2, b2, w3, b3, w4, b4)


def _sc_gather(T1, T2, src, dst):
    mesh = plsc.VectorSubcoreMesh(core_axis_name="c", subcore_axis_name="s")

    @functools.partial(
        pl.kernel,
        out_type=[jax.ShapeDtypeStruct((E, 128), _f32),
                  jax.ShapeDtypeStruct((E, 144), _f32)],
        mesh=mesh,
        compiler_params=pltpu.CompilerParams(use_tc_tiling_on_sc=False),
        scratch_types=[
            pltpu.VMEM((CH,), jnp.int32),
            pltpu.VMEM((CH,), jnp.int32),
            pltpu.VMEM((CH, 128), _f32),
            pltpu.VMEM((CH, 144), _f32),
            pltpu.SemaphoreType.DMA,
            pltpu.SemaphoreType.DMA,
        ],
    )
    def k(t1, t2, srcr, dstr, g1, g2, idx_s, idx_d, b1, b2, s1, s2):
        wid = lax.axis_index("s") * 2 + lax.axis_index("c")
        base = wid * EW

        def body(i, carry):
            off = base + i * CH
            pltpu.sync_copy(srcr.at[pl.ds(off, CH)], idx_s)
            pltpu.sync_copy(dstr.at[pl.ds(off, CH)], idx_d)
            c1 = pltpu.async_copy(t1.at[idx_s], b1, s1)
            c2 = pltpu.async_copy(t2.at[idx_d], b2, s2)
            c1.wait()
            c2.wait()
            pltpu.sync_copy(b1, g1.at[pl.ds(off, CH)])
            pltpu.sync_copy(b2, g2.at[pl.ds(off, CH)])
            return carry

        lax.fori_loop(0, NCH, body, 0)

    return k(T1, T2, src, dst)


def _sc_scatter(W, dst, zer):
    mesh = plsc.VectorSubcoreMesh(core_axis_name="c", subcore_axis_name="s")

    @functools.partial(
        pl.kernel,
        out_type=jax.ShapeDtypeStruct((2 * N, WD), _f32),
        mesh=mesh,
        compiler_params=pltpu.CompilerParams(use_tc_tiling_on_sc=False),
        scratch_types=[
            pltpu.VMEM((CH,), jnp.int32),
            pltpu.VMEM((CH, WD), _f32),
            pltpu.VMEM_SHARED((N, WD), _f32),
        ],
    )
    def k(w, dstr, z, acc_out, idx, wbuf, acc):
        cid = lax.axis_index("c")
        sid = lax.axis_index("s")
        base = (sid * 2 + cid) * EW
        row0 = sid * RT
        pltpu.sync_copy(z.at[pl.ds(row0, RT)], acc.at[pl.ds(row0, RT)])
        plsc.subcore_barrier()

        def body(i, carry):
            off = base + i * CH
            pltpu.sync_copy(dstr.at[pl.ds(off, CH)], idx)
            pltpu.sync_copy(w.at[pl.ds(off, CH)], wbuf)
            pltpu.sync_copy(wbuf, acc.at[idx], add=True)
            return carry

        lax.fori_loop(0, NCH, body, 0)
        plsc.subcore_barrier()
        pltpu.sync_copy(acc.at[pl.ds(row0, RT)],
                        acc_out.at[pl.ds(cid * N + row0, RT)])

    return k(W, dst, zer)


def kernel(x, edge_index, params):
    p = params
    src = edge_index[0]
    dst = edge_index[1]
    validf = (src != dst).astype(_f32).reshape(E, 1)
    zer = jnp.zeros((N, WD), _f32)

    def r1(v):
        return v.reshape(1, -1)

    h0 = _pre(x, p['pre1_W'], r1(p['bn_pre1_g']), r1(p['bn_pre1_b']),
              p['pre2_W'], r1(p['bn_pre2_g']), r1(p['bn_pre2_b']))
    hL = h0
    hR = h0
    for i in range(1, 5):
        attL = p['left%d_att' % i].reshape(1, F)
        attR = p['right%d_att' % i].reshape(1, F)
        T1, T2 = _tables(hL, hR,
                         p['left%d_Wl' % i], p['left%d_Wr' % i],
                         p['right%d_Wl' % i], p['right%d_Wr' % i],
                         attL, attR)
        G1, G2 = _sc_gather(T1, T2, src, dst)
        W = _edge(G1, G2, validf, attL, attR)
        ACC = _sc_scatter(W, dst, zer)
        hL = _combine(ACC[:, 0:80], T1[:, 0:64], hL,
                      r1(p['left%d_bng' % i]), r1(p['left%d_bnb' % i]), False)
        hR = _combine(ACC[:, 80:160], T1[:, 64:128], hR,
                      r1(p['right%d_bng' % i]), r1(p['right%d_bnb' % i]), True)
    return _post(hL, hR,
                 p['post1_W'], r1(p['post1_b']),
                 p['post2_W'], r1(p['post2_b']),
                 p['post3_W'], r1(p['post3_b']),
                 p['post4_W'], r1(p['post4_b']))
